# Initial kernel scaffold; baseline (speedup 1.0000x reference)
import jax
import jax.numpy as jnp
from jax.experimental import pallas as pl


def kernel(coords, grid0, grid1, grid2, grid3, grid4, grid5, grid6, grid7):
    raise NotImplementedError("write your pallas kernel here")


# speedup vs baseline: 180.9314x; 180.9314x over previous
import functools

import jax
import jax.numpy as jnp
from jax import lax
from jax.experimental import pallas as pl
from jax.experimental.pallas import tpu as pltpu
from jax.experimental.pallas import tpu_sc as plsc

RESOLUTIONS = (16, 32, 64, 128, 256, 512, 1024, 2048)
N_SMALL = 5
SMALL_OFF = (0, 256, 1280, 5376, 21760)
SMALL_TOT = 87296
BN = 524288
NC, NS = 2, 16
NW = NC * NS
NPT = BN // NW
C = 2048
NCH = NPT // C
SL = C // 16

_CLIP_HI = 1.0 - 1e-6


def _cell(xc, yc, r):
    rf = jnp.float32(r - 1)
    xs = xc * rf
    ys = yc * rf
    x0 = xs.astype(jnp.int32)
    y0 = ys.astype(jnp.int32)
    fx = xs - x0.astype(jnp.float32)
    fy = ys - y0.astype(jnp.float32)
    idx = y0 * r + x0
    return idx, fx, fy


def _lerp_packed(w00, w10, w01, w11, fx, fy):
    c00 = plsc.bitcast(w00, jnp.bfloat16)
    c10 = plsc.bitcast(w10, jnp.bfloat16)
    c01 = plsc.bitcast(w01, jnp.bfloat16)
    c11 = plsc.bitcast(w11, jnp.bfloat16)
    fxd = plsc.pack(fx, fx, format=plsc.PackFormat.INTERLEAVED)
    fyd = plsc.pack(fy, fy, format=plsc.PackFormat.INTERLEAVED)
    l0 = c00 + (c10 - c00) * fxd
    l1 = c01 + (c11 - c01) * fxd
    lf = l0 + (l1 - l0) * fyd
    f0, f1 = plsc.unpack(lf, format=plsc.PackFormat.INTERLEAVED)
    return f0, f1


@functools.partial(
    pl.kernel,
    out_type=(jax.ShapeDtypeStruct((BN,), jnp.float32),
              jax.ShapeDtypeStruct((BN,), jnp.float32)),
    mesh=plsc.VectorSubcoreMesh(core_axis_name="c", subcore_axis_name="s",
                                num_cores=NC, num_subcores=NS),
    scratch_types=[
        pltpu.VMEM((SMALL_TOT,), jnp.int32),
        pltpu.VMEM((C,), jnp.float32),
        pltpu.VMEM((C,), jnp.float32),
        pltpu.VMEM((C,), jnp.float32),
        pltpu.VMEM((C,), jnp.float32),
        pltpu.VMEM((C,), jnp.float32),
        pltpu.VMEM((C,), jnp.float32),
        pltpu.VMEM((4 * C,), jnp.int32),
        pltpu.VMEM((4 * C,), jnp.int32),
        pltpu.SemaphoreType.DMA,
    ],
    compiler_params=pltpu.CompilerParams(needs_layout_passes=False),
)
def _grid_kernel(x_hbm, y_hbm, small_hbm, t5_hbm, t6_hbm, t7_hbm,
                 out0_hbm, out1_hbm,
                 small_v, xv, yv, a0v, a1v, fxv, fyv, idxv, rowsv, sem):
    wid = lax.axis_index("s") * NC + lax.axis_index("c")
    base = wid * NPT
    pltpu.sync_copy(small_hbm, small_v)
    for g in range(NCH):
        cbase = base + g * C
        pltpu.sync_copy(x_hbm.at[pl.ds(cbase, C)], xv)
        pltpu.sync_copy(y_hbm.at[pl.ds(cbase, C)], yv)

        def small_body(s, carry):
            o = s * 16
            xc = jnp.clip(xv[pl.ds(o, 16)], 0.0, _CLIP_HI)
            yc = jnp.clip(yv[pl.ds(o, 16)], 0.0, _CLIP_HI)
            a0 = jnp.zeros((16,), jnp.float32)
            a1 = jnp.zeros((16,), jnp.float32)
            for l in range(N_SMALL):
                r = RESOLUTIONS[l]
                idx, fx, fy = _cell(xc, yc, r)
                b = idx + SMALL_OFF[l]
                w00 = plsc.load_gather(small_v, [b])
                w10 = plsc.load_gather(small_v, [b + 1])
                w01 = plsc.load_gather(small_v, [b + r])
                w11 = plsc.load_gather(small_v, [b + r + 1])
                f0, f1 = _lerp_packed(w00, w10, w01, w11, fx, fy)
                a0 = a0 + f0
                a1 = a1 + f1
            a0v[pl.ds(o, 16)] = a0
            a1v[pl.ds(o, 16)] = a1
            return carry

        lax.fori_loop(0, SL, small_body, 0)

        for l in range(N_SMALL, 8):
            r = RESOLUTIONS[l]
            t_hbm = (t5_hbm, t6_hbm, t7_hbm)[l - N_SMALL]

            def idx_body(s, carry, r=r):
                o = s * 16
                xc = jnp.clip(xv[pl.ds(o, 16)], 0.0, _CLIP_HI)
                yc = jnp.clip(yv[pl.ds(o, 16)], 0.0, _CLIP_HI)
                idx, fx, fy = _cell(xc, yc, r)
                idxv[pl.ds(o, 16)] = idx
                idxv[pl.ds(C + o, 16)] = idx + 1
                idxv[pl.ds(2 * C + o, 16)] = idx + r
                idxv[pl.ds(3 * C + o, 16)] = idx + r + 1
                fxv[pl.ds(o, 16)] = fx
                fyv[pl.ds(o, 16)] = fy
                return carry

            lax.fori_loop(0, SL, idx_body, 0)
            pltpu.async_copy(t_hbm.at[idxv], rowsv, sem).wait()

            def acc_body(s, carry):
                o = s * 16
                w00 = rowsv[pl.ds(o, 16)]
                w10 = rowsv[pl.ds(C + o, 16)]
                w01 = rowsv[pl.ds(2 * C + o, 16)]
                w11 = rowsv[pl.ds(3 * C + o, 16)]
                f0, f1 = _lerp_packed(w00, w10, w01, w11,
                                      fxv[pl.ds(o, 16)], fyv[pl.ds(o, 16)])
                a0v[pl.ds(o, 16)] = a0v[pl.ds(o, 16)] + f0
                a1v[pl.ds(o, 16)] = a1v[pl.ds(o, 16)] + f1
                return carry

            lax.fori_loop(0, SL, acc_body, 0)

        pltpu.sync_copy(a0v, out0_hbm.at[pl.ds(cbase, C)])
        pltpu.sync_copy(a1v, out1_hbm.at[pl.ds(cbase, C)])


def kernel(coords, grid0, grid1, grid2, grid3, grid4, grid5, grid6, grid7):
    grids = (grid0, grid1, grid2, grid3, grid4, grid5, grid6, grid7)
    ct = coords.T
    x = ct[0]
    y = ct[1]
    tabs = [lax.bitcast_convert_type(g.astype(jnp.bfloat16), jnp.int32)
            for g in grids]
    small = jnp.concatenate(tabs[:N_SMALL], axis=0)
    o0, o1 = _grid_kernel(x, y, small, tabs[5], tabs[6], tabs[7])
    return jnp.stack([o0, o1], axis=1).astype(jnp.float16)
